# Initial kernel scaffold; baseline (speedup 1.0000x reference)
import functools

import jax
import jax.numpy as jnp
from jax import lax
from jax.experimental import pallas as pl
from jax.experimental.pallas import tpu as pltpu
from jax.experimental.pallas import tpu_sc as plsc

NUM_TOKENS = 16384
SEQ = 50
DIM = 32
B = NUM_TOKENS * SEQ
NUM_WORKERS = 32
B_PER_W = B // NUM_WORKERS
CHUNK = 1024
N_CHUNKS = B_PER_W // CHUNK

_mesh = plsc.VectorSubcoreMesh(core_axis_name="c", subcore_axis_name="s")


@functools.partial(
    pl.kernel,
    mesh=_mesh,
    out_type=jax.ShapeDtypeStruct((B, DIM), jnp.float32),
    scratch_types=[
        pltpu.VMEM((CHUNK,), jnp.int32),
        pltpu.VMEM((CHUNK, DIM), jnp.float32),
        pltpu.SemaphoreType.DMA,
    ],
)
def _sc_gather(idx_hbm, table_hbm, out_hbm, idx_v, rows_v, sem):
    wid = lax.axis_index("s") * 2 + lax.axis_index("c")
    base = wid * B_PER_W

    def body(i, _):
        off = base + i * CHUNK
        pltpu.sync_copy(idx_hbm.at[pl.ds(off, CHUNK)], idx_v)
        pltpu.async_copy(table_hbm.at[idx_v], rows_v, sem).wait()
        pltpu.sync_copy(rows_v, out_hbm.at[pl.ds(off, CHUNK)])
        return ()

    lax.fori_loop(0, N_CHUNKS, body, ())


def kernel(token_ids, weight):
    flat = token_ids.reshape(-1)
    out = _sc_gather(flat, weight)
    return out.reshape(token_ids.shape + (weight.shape[1],))


# speedup vs baseline: 1.0950x; 1.0950x over previous
import functools

import jax
import jax.numpy as jnp
from jax import lax
from jax.experimental import pallas as pl
from jax.experimental.pallas import tpu as pltpu
from jax.experimental.pallas import tpu_sc as plsc

NUM_TOKENS = 16384
SEQ = 50
DIM = 32
B = NUM_TOKENS * SEQ
NUM_WORKERS = 32
B_PER_W = B // NUM_WORKERS
CHUNK = 1024
N_CHUNKS = B_PER_W // CHUNK

_mesh = plsc.VectorSubcoreMesh(core_axis_name="c", subcore_axis_name="s")


@functools.partial(
    pl.kernel,
    mesh=_mesh,
    out_type=jax.ShapeDtypeStruct((B, DIM), jnp.float32),
    scratch_types=[
        pltpu.VMEM((CHUNK,), jnp.int32),
        pltpu.VMEM((CHUNK, DIM), jnp.float32),
        pltpu.SemaphoreType.DMA,
    ],
    compiler_params=pltpu.CompilerParams(use_tc_tiling_on_sc=False),
)
def _sc_gather(idx_hbm, table_hbm, out_hbm, idx_v, rows_v, sem):
    wid = lax.axis_index("s") * 2 + lax.axis_index("c")
    base = wid * B_PER_W

    def body(i, _):
        off = base + i * CHUNK
        pltpu.sync_copy(idx_hbm.at[pl.ds(off, CHUNK)], idx_v)
        pltpu.async_copy(table_hbm.at[idx_v], rows_v, sem).wait()
        pltpu.sync_copy(rows_v, out_hbm.at[pl.ds(off, CHUNK)])
        return ()

    lax.fori_loop(0, N_CHUNKS, body, ())


def kernel(token_ids, weight):
    flat = token_ids.reshape(-1)
    out = _sc_gather(flat, weight)
    return out.reshape(token_ids.shape + (weight.shape[1],))


# speedup vs baseline: 1.7992x; 1.6432x over previous
import functools

import jax
import jax.numpy as jnp
from jax import lax
from jax.experimental import pallas as pl
from jax.experimental.pallas import tpu as pltpu
from jax.experimental.pallas import tpu_sc as plsc

NUM_TOKENS = 16384
SEQ = 50
DIM = 32
NUM_WORKERS = 32
ROWS_PER_W = NUM_TOKENS // NUM_WORKERS
R = 32
N_CHUNKS = ROWS_PER_W // R
NBUF = 2

_mesh = plsc.VectorSubcoreMesh(core_axis_name="c", subcore_axis_name="s")


@functools.partial(
    pl.kernel,
    mesh=_mesh,
    out_type=jax.ShapeDtypeStruct((NUM_TOKENS, SEQ, DIM), jnp.float32),
    scratch_types=[
        pltpu.VMEM((NBUF, R, SEQ), jnp.int32),
        pltpu.VMEM((NBUF, R, SEQ, DIM), jnp.float32),
        pltpu.SemaphoreType.DMA,
        pltpu.SemaphoreType.DMA,
        pltpu.SemaphoreType.DMA,
        pltpu.SemaphoreType.DMA,
        pltpu.SemaphoreType.DMA,
        pltpu.SemaphoreType.DMA,
    ],
    compiler_params=pltpu.CompilerParams(use_tc_tiling_on_sc=False),
)
def _sc_gather(idx_hbm, table_hbm, out_hbm, idx_v, rows_v,
               isem0, isem1, gsem0, gsem1, osem0, osem1):
    isem = (isem0, isem1)
    gsem = (gsem0, gsem1)
    osem = (osem0, osem1)
    wid = lax.axis_index("s") * 2 + lax.axis_index("c")
    row0 = wid * ROWS_PER_W

    def idx_src(g):
        return idx_hbm.at[pl.ds(row0 + g * R, R), :]

    def out_dst(g):
        return out_hbm.at[pl.ds(row0 + g * R, R), :, :]

    for b in range(NBUF):
        pltpu.async_copy(idx_src(b), idx_v.at[b], isem[b])

    def pair_body(p, _):
        for b in range(NBUF):
            g = p * NBUF + b
            pltpu.make_async_copy(
                idx_src(g), idx_v.at[b], isem[b]).wait()
            @pl.when(g >= NBUF)
            def _():
                pltpu.make_async_copy(
                    rows_v.at[b], out_dst(g - NBUF), osem[b]).wait()
            for j in range(R):
                pltpu.async_copy(
                    table_hbm.at[idx_v.at[b, j]],
                    rows_v.at[b, j], gsem[b])
            for j in range(R):
                pltpu.make_async_copy(
                    table_hbm.at[idx_v.at[b, j]],
                    rows_v.at[b, j], gsem[b]).wait()
            pltpu.async_copy(rows_v.at[b], out_dst(g), osem[b])
            @pl.when(g + NBUF < N_CHUNKS)
            def _():
                pltpu.async_copy(
                    idx_src(g + NBUF), idx_v.at[b], isem[b])
        return ()

    lax.fori_loop(0, N_CHUNKS // NBUF, pair_body, ())

    for b in range(NBUF):
        pltpu.make_async_copy(
            rows_v.at[b], out_dst(N_CHUNKS - NBUF + b), osem[b]).wait()


def kernel(token_ids, weight):
    return _sc_gather(token_ids, weight)
